# Initial kernel scaffold; baseline (speedup 1.0000x reference)
import jax
import jax.numpy as jnp
from jax.experimental import pallas as pl


def kernel(x, edge_index, Wl1, bl1, Wr1, Wl2, bl2, Wr2, Wl3, bl3, Wr3, Wreg, breg, Wd1, bd1, Wd2, bd2):
    raise NotImplementedError("write your pallas kernel here")


# speedup vs baseline: 5.4616x; 5.4616x over previous
import functools

import jax
import jax.numpy as jnp
from jax import lax
from jax.experimental import pallas as pl
from jax.experimental.pallas import tpu as pltpu
from jax.experimental.pallas import tpu_sc as plsc

N_NODES = 10000
NPAD = 10240
N_EDGES = 160000
K = 80
CHUNK_ROWS = N_EDGES // K
TILES = 16
ROWS_PER_TILE = CHUNK_ROWS // TILES
NODE_SLICE = NPAD // TILES


def _sc_agg(table0, table1, src2d, dst2d, zeros2d, zeros1d, feat, with_deg):
    mesh = plsc.VectorSubcoreMesh(core_axis_name="c", subcore_axis_name="s")
    out_type = [jax.ShapeDtypeStruct((NPAD, feat), jnp.float32),
                jax.ShapeDtypeStruct((NPAD, feat), jnp.float32)]
    scratch = [
        pltpu.VMEM((ROWS_PER_TILE, K), jnp.int32),
        pltpu.VMEM((ROWS_PER_TILE, K), jnp.int32),
        pltpu.VMEM((K, feat), jnp.float32),
        pltpu.VMEM_SHARED((NPAD, feat), jnp.float32),
        pltpu.SemaphoreType.DMA,
    ]
    if with_deg:
        out_type.append(jax.ShapeDtypeStruct((NPAD,), jnp.float32))
        scratch.append(pltpu.VMEM((K,), jnp.float32))
        scratch.append(pltpu.VMEM_SHARED((NPAD,), jnp.float32))

    def body(t0, t1, srcr, dstr, z2, z1, *refs):
        if with_deg:
            (o0, o1, degout, src_v, dst_v, rows_v, acc, sem,
             ones_v, degacc) = refs
        else:
            o0, o1, src_v, dst_v, rows_v, acc, sem = refs
        c = lax.axis_index("c")
        s = lax.axis_index("s")
        row0 = s * NODE_SLICE
        pltpu.sync_copy(z2.at[pl.ds(row0, NODE_SLICE)],
                        acc.at[pl.ds(row0, NODE_SLICE)])
        if with_deg:
            @pl.when(c == 0)
            def _():
                pltpu.sync_copy(z1.at[pl.ds(row0, NODE_SLICE)],
                                degacc.at[pl.ds(row0, NODE_SLICE)])
            for i in range(K // 16):
                ones_v[pl.ds(i * 16, 16)] = jnp.full((16,), 1.0, jnp.float32)
        pltpu.sync_copy(srcr.at[s], src_v)
        pltpu.sync_copy(dstr.at[s], dst_v)
        plsc.subcore_barrier()

        def run(table, do_deg):
            def step(j, carry):
                pltpu.async_copy(table.at[src_v.at[j]], rows_v, sem).wait()
                pltpu.sync_copy(rows_v, acc.at[dst_v.at[j]], add=True)
                if do_deg:
                    pltpu.sync_copy(ones_v, degacc.at[dst_v.at[j]], add=True)
                return carry
            lax.fori_loop(0, ROWS_PER_TILE, step, 0)

        pl.when(c == 0)(lambda: run(t0, with_deg))
        pl.when(c == 1)(lambda: run(t1, False))
        plsc.subcore_barrier()
        pl.when(c == 0)(lambda: pltpu.sync_copy(
            acc.at[pl.ds(row0, NODE_SLICE)], o0.at[pl.ds(row0, NODE_SLICE)]))
        pl.when(c == 1)(lambda: pltpu.sync_copy(
            acc.at[pl.ds(row0, NODE_SLICE)], o1.at[pl.ds(row0, NODE_SLICE)]))
        if with_deg:
            @pl.when(c == 0)
            def _():
                pltpu.sync_copy(degacc.at[pl.ds(row0, NODE_SLICE)],
                                degout.at[pl.ds(row0, NODE_SLICE)])

    run_kernel = pl.kernel(body, out_type=out_type, mesh=mesh,
                           scratch_types=scratch)
    return run_kernel(table0, table1, src2d, dst2d, zeros2d, zeros1d)


K_B = 100
ROWS_B = N_EDGES // (32 * K_B)


def _sc_agg_edgesplit(table, src3d, dst3d, zeros2d):
    mesh = plsc.VectorSubcoreMesh(core_axis_name="c", subcore_axis_name="s")
    out_type = [jax.ShapeDtypeStruct((NPAD, 128), jnp.float32),
                jax.ShapeDtypeStruct((NPAD, 128), jnp.float32)]
    scratch = [
        pltpu.VMEM((ROWS_B, K_B), jnp.int32),
        pltpu.VMEM((ROWS_B, K_B), jnp.int32),
        pltpu.VMEM((K_B, 128), jnp.float32),
        pltpu.VMEM_SHARED((NPAD, 128), jnp.float32),
        pltpu.SemaphoreType.DMA,
    ]

    def body(t, srcr, dstr, z2, o0, o1, src_v, dst_v, rows_v, acc, sem):
        c = lax.axis_index("c")
        s = lax.axis_index("s")
        q = c * TILES + s
        row0 = s * NODE_SLICE
        pltpu.sync_copy(z2.at[pl.ds(row0, NODE_SLICE)],
                        acc.at[pl.ds(row0, NODE_SLICE)])
        pltpu.sync_copy(srcr.at[q], src_v)
        pltpu.sync_copy(dstr.at[q], dst_v)
        plsc.subcore_barrier()

        def step(j, carry):
            pltpu.async_copy(t.at[src_v.at[j]], rows_v, sem).wait()
            pltpu.sync_copy(rows_v, acc.at[dst_v.at[j]], add=True)
            return carry
        lax.fori_loop(0, ROWS_B, step, 0)
        plsc.subcore_barrier()
        pl.when(c == 0)(lambda: pltpu.sync_copy(
            acc.at[pl.ds(row0, NODE_SLICE)], o0.at[pl.ds(row0, NODE_SLICE)]))
        pl.when(c == 1)(lambda: pltpu.sync_copy(
            acc.at[pl.ds(row0, NODE_SLICE)], o1.at[pl.ds(row0, NODE_SLICE)]))

    run_kernel = pl.kernel(body, out_type=out_type, mesh=mesh,
                           scratch_types=scratch)
    return run_kernel(table, src3d, dst3d, zeros2d)


def _dot_t(a, w):
    return lax.dot_general(a, w, (((1,), (1,)), ((), ())),
                           preferred_element_type=jnp.float32)


def _tc_layer_body(with_p, *refs):
    if with_p:
        (a0, a1, t0, t1, deg, wll, wlr, wrl, wrr, b, wp,
         o0, o1, po) = refs
    else:
        a0, a1, t0, t1, deg, wll, wlr, wrl, wrr, b, o0, o1 = refs
    inv = 1.0 / jnp.maximum(deg[...], 1.0)
    h = (_dot_t(a0[...] * inv, wll[...]) + _dot_t(a1[...] * inv, wlr[...])
         + _dot_t(t0[...], wrl[...]) + _dot_t(t1[...], wrr[...]) + b[...])
    h = jnp.maximum(h, 0.0)
    o0[...] = h[:, :128]
    o1[...] = h[:, 128:]
    if with_p:
        po[...] = _dot_t(h, wp[...])


def _tc_layer(a0, a1, t0, t1, deg, Wl, Wr, b, Wp=None):
    BN = 1024
    grid = (NPAD // BN,)
    half = pl.BlockSpec((BN, 128), lambda i: (i, 0))
    wspec = pl.BlockSpec((256, 128), lambda i: (0, 0))
    in_specs = [half, half, half, half,
                pl.BlockSpec((BN, 1), lambda i: (i, 0)),
                wspec, wspec, wspec, wspec,
                pl.BlockSpec((1, 256), lambda i: (0, 0))]
    out_shape = [jax.ShapeDtypeStruct((NPAD, 128), jnp.float32),
                 jax.ShapeDtypeStruct((NPAD, 128), jnp.float32)]
    out_specs = [half, half]
    args = [a0, a1, t0, t1, deg,
            Wl[:, :128], Wl[:, 128:], Wr[:, :128], Wr[:, 128:],
            b.reshape(1, 256)]
    if Wp is not None:
        in_specs.append(pl.BlockSpec((128, 256), lambda i: (0, 0)))
        out_shape.append(jax.ShapeDtypeStruct((NPAD, 128), jnp.float32))
        out_specs.append(half)
        args.append(Wp)
    return pl.pallas_call(
        functools.partial(_tc_layer_body, Wp is not None),
        grid=grid, in_specs=in_specs, out_specs=out_specs,
        out_shape=out_shape)(*args)


def _tc_final_body(a0, a1, t0, t1, deg, bl3, wr3l, wr3r, wreg, breg,
                   wd1, bd1, wd2, bd2, out):
    inv = 1.0 / jnp.maximum(deg[...], 1.0)
    m = (a0[...] + a1[...]) * inv
    h3 = m + bl3[...] + _dot_t(t0[...], wr3l[...]) + _dot_t(t1[...], wr3r[...])
    lat = _dot_t(h3, wreg[...]) + breg[...]
    d = jnp.maximum(_dot_t(lat, wd1[...]) + bd1[...], 0.0)
    out[...] = _dot_t(d, wd2[...]) + bd2[...]


def _tc_final(a0, a1, t0, t1, deg, bl3, Wr3, Wreg, breg, Wd1, bd1, Wd2, bd2):
    BN = 1024
    grid = (NPAD // BN,)
    in_specs = [pl.BlockSpec((BN, 128), lambda i: (i, 0)),
                pl.BlockSpec((BN, 128), lambda i: (i, 0)),
                pl.BlockSpec((BN, 128), lambda i: (i, 0)),
                pl.BlockSpec((BN, 128), lambda i: (i, 0)),
                pl.BlockSpec((BN, 1), lambda i: (i, 0)),
                pl.BlockSpec((1, 128), lambda i: (0, 0)),
                pl.BlockSpec((128, 128), lambda i: (0, 0)),
                pl.BlockSpec((128, 128), lambda i: (0, 0)),
                pl.BlockSpec((128, 128), lambda i: (0, 0)),
                pl.BlockSpec((1, 128), lambda i: (0, 0)),
                pl.BlockSpec((256, 128), lambda i: (0, 0)),
                pl.BlockSpec((1, 256), lambda i: (0, 0)),
                pl.BlockSpec((256, 256), lambda i: (0, 0)),
                pl.BlockSpec((1, 256), lambda i: (0, 0))]
    return pl.pallas_call(
        _tc_final_body, grid=grid, in_specs=in_specs,
        out_specs=pl.BlockSpec((BN, 256), lambda i: (i, 0)),
        out_shape=jax.ShapeDtypeStruct((NPAD, 256), jnp.float32),
    )(a0, a1, t0, t1, deg, bl3.reshape(1, 128),
      Wr3[:, :128], Wr3[:, 128:], Wreg, breg.reshape(1, 128),
      Wd1, bd1.reshape(1, 256), Wd2, bd2.reshape(1, 256))


def kernel(x, edge_index, Wl1, bl1, Wr1, Wl2, bl2, Wr2, Wl3, bl3, Wr3,
           Wreg, breg, Wd1, bd1, Wd2, bd2):
    ei = edge_index.astype(jnp.int32)
    src2d = ei[0].reshape(TILES, ROWS_PER_TILE, K)
    dst2d = ei[1].reshape(TILES, ROWS_PER_TILE, K)
    xp = jnp.pad(x, ((0, NPAD - N_NODES), (0, 0)))
    x0, x1 = xp[:, :128], xp[:, 128:]
    z128 = jnp.zeros((NPAD, 128), jnp.float32)
    z1 = jnp.zeros((NPAD,), jnp.float32)

    seg1_0, seg1_1, deg = _sc_agg(x0, x1, src2d, dst2d, z128, z1,
                                  feat=128, with_deg=True)
    deg = deg.reshape(NPAD, 1)
    h1_0, h1_1 = _tc_layer(seg1_0, seg1_1, x0, x1, deg, Wl1, Wr1, bl1)
    seg2_0, seg2_1 = _sc_agg(h1_0, h1_1, src2d, dst2d, z128, z1,
                             feat=128, with_deg=False)
    h2_0, h2_1, p = _tc_layer(seg2_0, seg2_1, h1_0, h1_1, deg,
                              Wl2, Wr2, bl2, Wp=Wl3)
    src3b = ei[0].reshape(32, ROWS_B, K_B)
    dst3b = ei[1].reshape(32, ROWS_B, K_B)
    seg3_0, seg3_1 = _sc_agg_edgesplit(p, src3b, dst3b, z128)
    out = _tc_final(seg3_0, seg3_1, h2_0, h2_1, deg, bl3, Wr3,
                    Wreg, breg, Wd1, bd1, Wd2, bd2)
    return out[:N_NODES]
